# baseline (speedup 1.0000x reference)
import functools

import jax
import jax.numpy as jnp
from jax import lax
from jax.experimental import pallas as pl
from jax.experimental.pallas import tpu as pltpu
from jax.experimental.pallas import tpu_sc as plsc

BS, C, NY, NX = 8, 192, 64, 64
NC, NA = 80, 1
K = 8192
NPIX = BS * NY * NX
BLK = 256
STEM_ROWS = NPIX + BLK
PB = (NY * NX) // BLK

NWORK = 32
BPW = K // NWORK
GCH = 128
NCH = BPW // GCH

KB = 512
OUT_W = 4 + 1 + NC


def _stem_body(x_ref, w_ref, b_ref, o_ref):
    i = pl.program_id(0)
    last = pl.num_programs(0) - 1

    @pl.when(i == last)
    def _():
        o_ref[...] = jnp.zeros_like(o_ref)

    @pl.when(i < last)
    def _():
        acc = lax.dot_general(x_ref[0], w_ref[...], (((0,), (1,)), ((), ())),
                              preferred_element_type=jnp.float32)
        acc = acc + b_ref[...]
        o_ref[...] = acc * jax.nn.sigmoid(acc)


def _stem(x3, stem_w, stem_b2):
    nblk = STEM_ROWS // BLK
    cap = NPIX // BLK - 1
    return pl.pallas_call(
        _stem_body,
        grid=(nblk,),
        in_specs=[
            pl.BlockSpec((1, C, BLK),
                         lambda i: (jnp.minimum(i, cap) // PB, 0,
                                    jnp.minimum(i, cap) % PB)),
            pl.BlockSpec((C, C), lambda i: (0, 0)),
            pl.BlockSpec((1, C), lambda i: (0, 0)),
        ],
        out_specs=pl.BlockSpec((BLK, C), lambda i: (i, 0)),
        out_shape=jax.ShapeDtypeStruct((STEM_ROWS, C), jnp.float32),
    )(x3, stem_w, stem_b2)


_TAPS = [(dy, dx) for dy in (-1, 0, 1) for dx in (-1, 0, 1)]


def _gather_body(stem_hbm, bi_hbm, yi_hbm, xi_hbm, g_hbm, *refs):
    (bv, yv, xv), taps = refs[0:3], refs[3:12]
    bufs, gsem, wsem = refs[12:14], refs[14:16], refs[16:18]
    wid = lax.axis_index("s") * 2 + lax.axis_index("c")
    base = wid * BPW
    pltpu.sync_copy(bi_hbm.at[pl.ds(base, BPW)], bv)
    pltpu.sync_copy(yi_hbm.at[pl.ds(base, BPW)], yv)
    pltpu.sync_copy(xi_hbm.at[pl.ds(base, BPW)], xv)

    for j in range(BPW // 16):
        sl = pl.ds(j * 16, 16)
        b = bv[sl]
        y = yv[sl]
        x = xv[sl]
        flat = (b * NY + y) * NX + x
        y_lo = y >= 1
        y_hi = y <= NY - 2
        x_lo = x >= 1
        x_hi = x <= NX - 2
        for t, (dy, dx) in enumerate(_TAPS):
            ok = None
            if dy == -1:
                ok = y_lo
            elif dy == 1:
                ok = y_hi
            if dx == -1:
                ok = x_lo if ok is None else (ok & x_lo)
            elif dx == 1:
                ok = x_hi if ok is None else (ok & x_hi)
            ft = flat + (dy * NX + dx)
            if ok is not None:
                ft = jnp.where(ok, ft, NPIX)
            taps[t][sl] = ft

    chunks = [(t, cc) for t in range(9) for cc in range(NCH)]
    n = len(chunks)

    def _start_gather(i, b):
        t, cc = chunks[i]
        return pltpu.async_copy(
            stem_hbm.at[taps[t].at[pl.ds(cc * GCH, GCH)]], bufs[b], gsem[b])

    def _start_write(i, b):
        t, cc = chunks[i]
        return pltpu.async_copy(
            bufs[b], g_hbm.at[t, pl.ds(base + cc * GCH, GCH)], wsem[b])

    gdesc = [None, None]
    wdesc = [None, None]
    gdesc[0] = _start_gather(0, 0)
    for i in range(n):
        b = i % 2
        if i + 1 < n:
            nb = (i + 1) % 2
            if wdesc[nb] is not None:
                wdesc[nb].wait()
            gdesc[nb] = _start_gather(i + 1, nb)
        gdesc[b].wait()
        wdesc[b] = _start_write(i, b)
    wdesc[(n - 1) % 2].wait()
    wdesc[n % 2].wait()


def _gather(stem_full, bi, yi, xi):
    mesh = plsc.VectorSubcoreMesh(core_axis_name="c", subcore_axis_name="s")
    return pl.kernel(
        _gather_body,
        out_type=jax.ShapeDtypeStruct((9, K, C), jnp.float32),
        mesh=mesh,
        compiler_params=pltpu.CompilerParams(use_tc_tiling_on_sc=False),
        scratch_types=(
            [pltpu.VMEM((BPW,), jnp.int32) for _ in range(3)]
            + [pltpu.VMEM((BPW,), jnp.int32) for _ in range(9)]
            + [pltpu.VMEM((GCH, C), jnp.float32) for _ in range(2)]
            + [pltpu.SemaphoreType.DMA for _ in range(4)]
        ),
    )(stem_full, bi, yi, xi)


def _head_body(g_ref, wc_ref, wr_ref, cb_ref, rb_ref, wro_ref, wcb_ref,
               hb_ref, o_ref):
    accc = jnp.zeros((KB, C), jnp.float32) + cb_ref[...]
    accr = jnp.zeros((KB, C), jnp.float32) + rb_ref[...]
    for t in range(9):
        g = g_ref[t]
        accc += lax.dot_general(g, wc_ref[t], (((1,), (1,)), ((), ())),
                                preferred_element_type=jnp.float32)
        accr += lax.dot_general(g, wr_ref[t], (((1,), (1,)), ((), ())),
                                preferred_element_type=jnp.float32)
    cls_f = accc * jax.nn.sigmoid(accc)
    reg_f = accr * jax.nn.sigmoid(accr)
    out = lax.dot_general(reg_f, wro_ref[...], (((1,), (1,)), ((), ())),
                          preferred_element_type=jnp.float32)
    out += lax.dot_general(cls_f, wcb_ref[...], (((1,), (1,)), ((), ())),
                           preferred_element_type=jnp.float32)
    o_ref[...] = out + hb_ref[...]


def _heads(g, w9c, w9r, cb2, rb2, wro, wcb, hbias):
    return pl.pallas_call(
        _head_body,
        grid=(K // KB,),
        in_specs=[
            pl.BlockSpec((9, KB, C), lambda i: (0, i, 0)),
            pl.BlockSpec((9, C, C), lambda i: (0, 0, 0)),
            pl.BlockSpec((9, C, C), lambda i: (0, 0, 0)),
            pl.BlockSpec((1, C), lambda i: (0, 0)),
            pl.BlockSpec((1, C), lambda i: (0, 0)),
            pl.BlockSpec((OUT_W, C), lambda i: (0, 0)),
            pl.BlockSpec((OUT_W, C), lambda i: (0, 0)),
            pl.BlockSpec((1, OUT_W), lambda i: (0, 0)),
        ],
        out_specs=pl.BlockSpec((KB, OUT_W), lambda i: (i, 0)),
        out_shape=jax.ShapeDtypeStruct((K, OUT_W), jnp.float32),
    )(g, w9c, w9r, cb2, rb2, wro, wcb, hbias)


def kernel(x, indices, stem_w, stem_b, cls_conv_w, cls_conv_b,
           reg_conv_w, reg_conv_b, cls_pred_w, cls_pred_b,
           reg_pred_w, reg_pred_b, obj_pred_w, obj_pred_b):
    x3 = x.reshape(BS, C, NY * NX)
    stem_full = _stem(x3, stem_w, stem_b.reshape(1, C))

    idx32 = indices.astype(jnp.int32)
    g = _gather(stem_full, idx32[:, 0], idx32[:, 1], idx32[:, 2])

    w9c = jnp.transpose(cls_conv_w.reshape(C, C, 9), (2, 0, 1))
    w9r = jnp.transpose(reg_conv_w.reshape(C, C, 9), (2, 0, 1))
    wro = jnp.concatenate(
        [reg_pred_w, obj_pred_w, jnp.zeros((NC, C), jnp.float32)], axis=0)
    wcb = jnp.concatenate(
        [jnp.zeros((4 + 1, C), jnp.float32), cls_pred_w], axis=0)
    hbias = jnp.concatenate(
        [reg_pred_b, obj_pred_b, cls_pred_b]).reshape(1, OUT_W)

    return _heads(g, w9c, w9r, cls_conv_b.reshape(1, C),
                  reg_conv_b.reshape(1, C), wro, wcb, hbias)


# speedup vs baseline: 1.2290x; 1.2290x over previous
import functools

import jax
import jax.numpy as jnp
from jax import lax
from jax.experimental import pallas as pl
from jax.experimental.pallas import tpu as pltpu
from jax.experimental.pallas import tpu_sc as plsc

BS, C, NY, NX = 8, 192, 64, 64
NC, NA = 80, 1
K = 8192
NPIX = BS * NY * NX
BLK = 256
STEM_ROWS = NPIX + BLK
PB = (NY * NX) // BLK

NWORK = 32
BPW = K // NWORK
GCH = 128
NCH = BPW // GCH

KB = 512
OUT_W = 4 + 1 + NC
CP = 256


def _stem_body(x_ref, w_ref, b_ref, o_ref):
    i = pl.program_id(0)
    last = pl.num_programs(0) - 1

    @pl.when(i == last)
    def _():
        o_ref[...] = jnp.zeros_like(o_ref)

    @pl.when(i < last)
    def _():
        acc = lax.dot_general(x_ref[0], w_ref[...], (((0,), (1,)), ((), ())),
                              preferred_element_type=jnp.float32)
        acc = acc + b_ref[...]
        o_ref[...] = acc * jax.nn.sigmoid(acc)


def _stem(x3, stem_w, stem_b2):
    nblk = STEM_ROWS // BLK
    cap = NPIX // BLK - 1
    return pl.pallas_call(
        _stem_body,
        grid=(nblk,),
        in_specs=[
            pl.BlockSpec((1, C, BLK),
                         lambda i: (jnp.minimum(i, cap) // PB, 0,
                                    jnp.minimum(i, cap) % PB)),
            pl.BlockSpec((CP, C), lambda i: (0, 0)),
            pl.BlockSpec((1, CP), lambda i: (0, 0)),
        ],
        out_specs=pl.BlockSpec((BLK, CP), lambda i: (i, 0)),
        out_shape=jax.ShapeDtypeStruct((STEM_ROWS, CP), jnp.float32),
    )(x3, stem_w, stem_b2)


_TAPS = [(dy, dx) for dy in (-1, 0, 1) for dx in (-1, 0, 1)]


def _gather_body(stem_hbm, bi_hbm, yi_hbm, xi_hbm, g_hbm, *refs):
    (bv, yv, xv), taps = refs[0:3], refs[3:12]
    bufs, gsem, wsem = refs[12:14], refs[14:16], refs[16:18]
    wid = lax.axis_index("s") * 2 + lax.axis_index("c")
    base = wid * BPW
    pltpu.sync_copy(bi_hbm.at[pl.ds(base, BPW)], bv)
    pltpu.sync_copy(yi_hbm.at[pl.ds(base, BPW)], yv)
    pltpu.sync_copy(xi_hbm.at[pl.ds(base, BPW)], xv)

    for j in range(BPW // 16):
        sl = pl.ds(j * 16, 16)
        b = bv[sl]
        y = yv[sl]
        x = xv[sl]
        flat = (b * NY + y) * NX + x
        y_lo = y >= 1
        y_hi = y <= NY - 2
        x_lo = x >= 1
        x_hi = x <= NX - 2
        for t, (dy, dx) in enumerate(_TAPS):
            ok = None
            if dy == -1:
                ok = y_lo
            elif dy == 1:
                ok = y_hi
            if dx == -1:
                ok = x_lo if ok is None else (ok & x_lo)
            elif dx == 1:
                ok = x_hi if ok is None else (ok & x_hi)
            ft = flat + (dy * NX + dx)
            if ok is not None:
                ft = jnp.where(ok, ft, NPIX)
            taps[t][sl] = ft

    chunks = [(t, cc) for t in range(9) for cc in range(NCH)]
    n = len(chunks)

    def _start_gather(i, b):
        t, cc = chunks[i]
        return pltpu.async_copy(
            stem_hbm.at[taps[t].at[pl.ds(cc * GCH, GCH)]], bufs[b], gsem[b])

    def _start_write(i, b):
        t, cc = chunks[i]
        return pltpu.async_copy(
            bufs[b], g_hbm.at[t, pl.ds(base + cc * GCH, GCH)], wsem[b])

    gdesc = [None, None]
    wdesc = [None, None]
    gdesc[0] = _start_gather(0, 0)
    for i in range(n):
        b = i % 2
        if i + 1 < n:
            nb = (i + 1) % 2
            if wdesc[nb] is not None:
                wdesc[nb].wait()
            gdesc[nb] = _start_gather(i + 1, nb)
        gdesc[b].wait()
        wdesc[b] = _start_write(i, b)
    wdesc[(n - 1) % 2].wait()
    wdesc[n % 2].wait()


def _gather(stem_full, bi, yi, xi):
    mesh = plsc.VectorSubcoreMesh(core_axis_name="c", subcore_axis_name="s")
    return pl.kernel(
        _gather_body,
        out_type=jax.ShapeDtypeStruct((9, K, CP), jnp.float32),
        mesh=mesh,
        scratch_types=(
            [pltpu.VMEM((BPW,), jnp.int32) for _ in range(3)]
            + [pltpu.VMEM((BPW,), jnp.int32) for _ in range(9)]
            + [pltpu.VMEM((GCH, CP), jnp.float32) for _ in range(2)]
            + [pltpu.SemaphoreType.DMA for _ in range(4)]
        ),
    )(stem_full, bi, yi, xi)


def _head_body(g_ref, wc_ref, wr_ref, cb_ref, rb_ref, wro_ref, wcb_ref,
               hb_ref, o_ref):
    accc = jnp.zeros((KB, C), jnp.float32) + cb_ref[...]
    accr = jnp.zeros((KB, C), jnp.float32) + rb_ref[...]
    for t in range(9):
        g = g_ref[t]
        accc += lax.dot_general(g, wc_ref[t], (((1,), (1,)), ((), ())),
                                preferred_element_type=jnp.float32)
        accr += lax.dot_general(g, wr_ref[t], (((1,), (1,)), ((), ())),
                                preferred_element_type=jnp.float32)
    cls_f = accc * jax.nn.sigmoid(accc)
    reg_f = accr * jax.nn.sigmoid(accr)
    out = lax.dot_general(reg_f, wro_ref[...], (((1,), (1,)), ((), ())),
                          preferred_element_type=jnp.float32)
    out += lax.dot_general(cls_f, wcb_ref[...], (((1,), (1,)), ((), ())),
                           preferred_element_type=jnp.float32)
    o_ref[...] = out + hb_ref[...]


def _heads(g, w9c, w9r, cb2, rb2, wro, wcb, hbias):
    return pl.pallas_call(
        _head_body,
        grid=(K // KB,),
        in_specs=[
            pl.BlockSpec((9, KB, CP), lambda i: (0, i, 0)),
            pl.BlockSpec((9, C, CP), lambda i: (0, 0, 0)),
            pl.BlockSpec((9, C, CP), lambda i: (0, 0, 0)),
            pl.BlockSpec((1, C), lambda i: (0, 0)),
            pl.BlockSpec((1, C), lambda i: (0, 0)),
            pl.BlockSpec((OUT_W, C), lambda i: (0, 0)),
            pl.BlockSpec((OUT_W, C), lambda i: (0, 0)),
            pl.BlockSpec((1, OUT_W), lambda i: (0, 0)),
        ],
        out_specs=pl.BlockSpec((KB, OUT_W), lambda i: (i, 0)),
        out_shape=jax.ShapeDtypeStruct((K, OUT_W), jnp.float32),
    )(g, w9c, w9r, cb2, rb2, wro, wcb, hbias)


def kernel(x, indices, stem_w, stem_b, cls_conv_w, cls_conv_b,
           reg_conv_w, reg_conv_b, cls_pred_w, cls_pred_b,
           reg_pred_w, reg_pred_b, obj_pred_w, obj_pred_b):
    x3 = x.reshape(BS, C, NY * NX)
    stem_wp = jnp.pad(stem_w, ((0, CP - C), (0, 0)))
    stem_bp = jnp.pad(stem_b, (0, CP - C)).reshape(1, CP)
    stem_full = _stem(x3, stem_wp, stem_bp)

    idx32 = indices.astype(jnp.int32)
    g = _gather(stem_full, idx32[:, 0], idx32[:, 1], idx32[:, 2])

    w9c = jnp.pad(jnp.transpose(cls_conv_w.reshape(C, C, 9), (2, 0, 1)),
                  ((0, 0), (0, 0), (0, CP - C)))
    w9r = jnp.pad(jnp.transpose(reg_conv_w.reshape(C, C, 9), (2, 0, 1)),
                  ((0, 0), (0, 0), (0, CP - C)))
    wro = jnp.concatenate(
        [reg_pred_w, obj_pred_w, jnp.zeros((NC, C), jnp.float32)], axis=0)
    wcb = jnp.concatenate(
        [jnp.zeros((4 + 1, C), jnp.float32), cls_pred_w], axis=0)
    hbias = jnp.concatenate(
        [reg_pred_b, obj_pred_b, cls_pred_b]).reshape(1, OUT_W)

    return _heads(g, w9c, w9r, cls_conv_b.reshape(1, C),
                  reg_conv_b.reshape(1, C), wro, wcb, hbias)


# speedup vs baseline: 1.5845x; 1.2892x over previous
import functools

import jax
import jax.numpy as jnp
from jax import lax
from jax.experimental import pallas as pl
from jax.experimental.pallas import tpu as pltpu
from jax.experimental.pallas import tpu_sc as plsc

BS, C, NY, NX = 8, 192, 64, 64
NC, NA = 80, 1
K = 8192
NPIX = BS * NY * NX
BLK = 1024
STEM_ROWS = NPIX + BLK

NWORK = 32
BPW = K // NWORK
GCH = 128
NCH = BPW // GCH

KB = 512
OUT_W = 4 + 1 + NC
CP = 256


def _stem_body(x_ref, w_ref, b_ref, o_ref):
    i = pl.program_id(0)
    last = pl.num_programs(0) - 1

    @pl.when(i == last)
    def _():
        o_ref[...] = jnp.zeros_like(o_ref)

    @pl.when(i < last)
    def _():
        acc = lax.dot_general(x_ref[...], w_ref[...], (((1,), (0,)), ((), ())),
                              preferred_element_type=jnp.float32)
        acc = acc + b_ref[...]
        o_ref[...] = acc * jax.nn.sigmoid(acc)


def _stem(xt, stem_wt, stem_b2):
    nblk = STEM_ROWS // BLK
    cap = NPIX // BLK - 1
    return pl.pallas_call(
        _stem_body,
        grid=(nblk,),
        in_specs=[
            pl.BlockSpec((BLK, C), lambda i: (jnp.minimum(i, cap), 0)),
            pl.BlockSpec((C, CP), lambda i: (0, 0)),
            pl.BlockSpec((1, CP), lambda i: (0, 0)),
        ],
        out_specs=pl.BlockSpec((BLK, CP), lambda i: (i, 0)),
        out_shape=jax.ShapeDtypeStruct((STEM_ROWS, CP), jnp.float32),
    )(xt, stem_wt, stem_b2)


_TAPS = [(dy, dx) for dy in (-1, 0, 1) for dx in (-1, 0, 1)]


def _gather_body(stem_hbm, bi_hbm, yi_hbm, xi_hbm, g_hbm, *refs):
    (bv, yv, xv), taps = refs[0:3], refs[3:12]
    bufs, gsem, wsem = refs[12:14], refs[14:16], refs[16:18]
    wid = lax.axis_index("s") * 2 + lax.axis_index("c")
    base = wid * BPW
    pltpu.sync_copy(bi_hbm.at[pl.ds(base, BPW)], bv)
    pltpu.sync_copy(yi_hbm.at[pl.ds(base, BPW)], yv)
    pltpu.sync_copy(xi_hbm.at[pl.ds(base, BPW)], xv)

    for j in range(BPW // 16):
        sl = pl.ds(j * 16, 16)
        b = bv[sl]
        y = yv[sl]
        x = xv[sl]
        flat = (b * NY + y) * NX + x
        y_lo = y >= 1
        y_hi = y <= NY - 2
        x_lo = x >= 1
        x_hi = x <= NX - 2
        for t, (dy, dx) in enumerate(_TAPS):
            ok = None
            if dy == -1:
                ok = y_lo
            elif dy == 1:
                ok = y_hi
            if dx == -1:
                ok = x_lo if ok is None else (ok & x_lo)
            elif dx == 1:
                ok = x_hi if ok is None else (ok & x_hi)
            ft = flat + (dy * NX + dx)
            if ok is not None:
                ft = jnp.where(ok, ft, NPIX)
            taps[t][sl] = ft

    chunks = [(t, cc) for t in range(9) for cc in range(NCH)]
    n = len(chunks)

    def _start_gather(i, b):
        t, cc = chunks[i]
        return pltpu.async_copy(
            stem_hbm.at[taps[t].at[pl.ds(cc * GCH, GCH)]], bufs[b], gsem[b])

    def _start_write(i, b):
        t, cc = chunks[i]
        return pltpu.async_copy(
            bufs[b], g_hbm.at[t, pl.ds(base + cc * GCH, GCH)], wsem[b])

    gdesc = [None, None]
    wdesc = [None, None]
    gdesc[0] = _start_gather(0, 0)
    for i in range(n):
        b = i % 2
        if i + 1 < n:
            nb = (i + 1) % 2
            if wdesc[nb] is not None:
                wdesc[nb].wait()
            gdesc[nb] = _start_gather(i + 1, nb)
        gdesc[b].wait()
        wdesc[b] = _start_write(i, b)
    wdesc[(n - 1) % 2].wait()
    wdesc[n % 2].wait()


def _gather(stem_full, bi, yi, xi):
    mesh = plsc.VectorSubcoreMesh(core_axis_name="c", subcore_axis_name="s")
    return pl.kernel(
        _gather_body,
        out_type=jax.ShapeDtypeStruct((9, K, CP), jnp.float32),
        mesh=mesh,
        scratch_types=(
            [pltpu.VMEM((BPW,), jnp.int32) for _ in range(3)]
            + [pltpu.VMEM((BPW,), jnp.int32) for _ in range(9)]
            + [pltpu.VMEM((GCH, CP), jnp.float32) for _ in range(2)]
            + [pltpu.SemaphoreType.DMA for _ in range(4)]
        ),
    )(stem_full, bi, yi, xi)


def _head_body(g_ref, wc_ref, wr_ref, cb_ref, rb_ref, wro_ref, wcb_ref,
               hb_ref, o_ref):
    accc = jnp.zeros((KB, C), jnp.float32) + cb_ref[...]
    accr = jnp.zeros((KB, C), jnp.float32) + rb_ref[...]
    for t in range(9):
        g = g_ref[t]
        accc += lax.dot_general(g, wc_ref[t], (((1,), (1,)), ((), ())),
                                preferred_element_type=jnp.float32)
        accr += lax.dot_general(g, wr_ref[t], (((1,), (1,)), ((), ())),
                                preferred_element_type=jnp.float32)
    cls_f = accc * jax.nn.sigmoid(accc)
    reg_f = accr * jax.nn.sigmoid(accr)
    out = lax.dot_general(reg_f, wro_ref[...], (((1,), (1,)), ((), ())),
                          preferred_element_type=jnp.float32)
    out += lax.dot_general(cls_f, wcb_ref[...], (((1,), (1,)), ((), ())),
                           preferred_element_type=jnp.float32)
    o_ref[...] = out + hb_ref[...]


def _heads(g, w9c, w9r, cb2, rb2, wro, wcb, hbias):
    return pl.pallas_call(
        _head_body,
        grid=(K // KB,),
        in_specs=[
            pl.BlockSpec((9, KB, CP), lambda i: (0, i, 0)),
            pl.BlockSpec((9, C, CP), lambda i: (0, 0, 0)),
            pl.BlockSpec((9, C, CP), lambda i: (0, 0, 0)),
            pl.BlockSpec((1, C), lambda i: (0, 0)),
            pl.BlockSpec((1, C), lambda i: (0, 0)),
            pl.BlockSpec((OUT_W, C), lambda i: (0, 0)),
            pl.BlockSpec((OUT_W, C), lambda i: (0, 0)),
            pl.BlockSpec((1, OUT_W), lambda i: (0, 0)),
        ],
        out_specs=pl.BlockSpec((KB, OUT_W), lambda i: (i, 0)),
        out_shape=jax.ShapeDtypeStruct((K, OUT_W), jnp.float32),
    )(g, w9c, w9r, cb2, rb2, wro, wcb, hbias)


def kernel(x, indices, stem_w, stem_b, cls_conv_w, cls_conv_b,
           reg_conv_w, reg_conv_b, cls_pred_w, cls_pred_b,
           reg_pred_w, reg_pred_b, obj_pred_w, obj_pred_b):
    xt = jnp.transpose(x.reshape(BS, C, NY * NX), (0, 2, 1)).reshape(NPIX, C)
    stem_wt = jnp.pad(stem_w, ((0, CP - C), (0, 0))).T
    stem_bp = jnp.pad(stem_b, (0, CP - C)).reshape(1, CP)
    stem_full = _stem(xt, stem_wt, stem_bp)

    idx32 = indices.astype(jnp.int32)
    g = _gather(stem_full, idx32[:, 0], idx32[:, 1], idx32[:, 2])

    w9c = jnp.pad(jnp.transpose(cls_conv_w.reshape(C, C, 9), (2, 0, 1)),
                  ((0, 0), (0, 0), (0, CP - C)))
    w9r = jnp.pad(jnp.transpose(reg_conv_w.reshape(C, C, 9), (2, 0, 1)),
                  ((0, 0), (0, 0), (0, CP - C)))
    wro = jnp.concatenate(
        [reg_pred_w, obj_pred_w, jnp.zeros((NC, C), jnp.float32)], axis=0)
    wcb = jnp.concatenate(
        [jnp.zeros((4 + 1, C), jnp.float32), cls_pred_w], axis=0)
    hbias = jnp.concatenate(
        [reg_pred_b, obj_pred_b, cls_pred_b]).reshape(1, OUT_W)

    return _heads(g, w9c, w9r, cls_conv_b.reshape(1, C),
                  reg_conv_b.reshape(1, C), wro, wcb, hbias)


# speedup vs baseline: 1.8006x; 1.1364x over previous
import functools

import jax
import jax.numpy as jnp
from jax import lax
from jax.experimental import pallas as pl
from jax.experimental.pallas import tpu as pltpu
from jax.experimental.pallas import tpu_sc as plsc

BS, C, NY, NX = 8, 192, 64, 64
NC, NA = 80, 1
K = 8192
NPIX = BS * NY * NX
BLK = 1024
STEM_ROWS = NPIX + BLK

NWORK = 32
BPW = K // NWORK
GCH = 128
NCH = BPW // GCH

KB = 512
OUT_W = 4 + 1 + NC
CP = 256
CH = 128


def _stem_body(x_ref, w_ref, b_ref, o_ref):
    i = pl.program_id(0)
    last = pl.num_programs(0) - 1

    @pl.when(i == last)
    def _():
        o_ref[...] = jnp.zeros_like(o_ref)

    @pl.when(i < last)
    def _():
        acc = lax.dot_general(x_ref[...], w_ref[...], (((1,), (0,)), ((), ())),
                              preferred_element_type=jnp.float32)
        acc = acc + b_ref[...]
        act = acc * jax.nn.sigmoid(acc)
        lo = lax.bitcast_convert_type(
            act[:, :CH].astype(jnp.bfloat16), jnp.uint16).astype(jnp.uint32)
        hi = lax.bitcast_convert_type(
            act[:, CH:].astype(jnp.bfloat16), jnp.uint16).astype(jnp.uint32)
        o_ref[...] = lax.bitcast_convert_type((hi << 16) | lo, jnp.int32)


def _stem(xt, stem_wt, stem_b2):
    nblk = STEM_ROWS // BLK
    cap = NPIX // BLK - 1
    return pl.pallas_call(
        _stem_body,
        grid=(nblk,),
        in_specs=[
            pl.BlockSpec((BLK, C), lambda i: (jnp.minimum(i, cap), 0)),
            pl.BlockSpec((C, CP), lambda i: (0, 0)),
            pl.BlockSpec((1, CP), lambda i: (0, 0)),
        ],
        out_specs=pl.BlockSpec((BLK, CH), lambda i: (i, 0)),
        out_shape=jax.ShapeDtypeStruct((STEM_ROWS, CH), jnp.int32),
    )(xt, stem_wt, stem_b2)


_TAPS = [(dy, dx) for dy in (-1, 0, 1) for dx in (-1, 0, 1)]


def _gather_body(stem_hbm, bi_hbm, yi_hbm, xi_hbm, g_hbm, *refs):
    (bv, yv, xv), taps = refs[0:3], refs[3:12]
    bufs, gsem, wsem = refs[12:14], refs[14:16], refs[16:18]
    wid = lax.axis_index("s") * 2 + lax.axis_index("c")
    base = wid * BPW
    pltpu.sync_copy(bi_hbm.at[pl.ds(base, BPW)], bv)
    pltpu.sync_copy(yi_hbm.at[pl.ds(base, BPW)], yv)
    pltpu.sync_copy(xi_hbm.at[pl.ds(base, BPW)], xv)

    for j in range(BPW // 16):
        sl = pl.ds(j * 16, 16)
        b = bv[sl]
        y = yv[sl]
        x = xv[sl]
        flat = (b * NY + y) * NX + x
        y_lo = y >= 1
        y_hi = y <= NY - 2
        x_lo = x >= 1
        x_hi = x <= NX - 2
        for t, (dy, dx) in enumerate(_TAPS):
            ok = None
            if dy == -1:
                ok = y_lo
            elif dy == 1:
                ok = y_hi
            if dx == -1:
                ok = x_lo if ok is None else (ok & x_lo)
            elif dx == 1:
                ok = x_hi if ok is None else (ok & x_hi)
            ft = flat + (dy * NX + dx)
            if ok is not None:
                ft = jnp.where(ok, ft, NPIX)
            taps[t][sl] = ft

    chunks = [(t, cc) for t in range(9) for cc in range(NCH)]
    n = len(chunks)

    def _start_gather(i, b):
        t, cc = chunks[i]
        return pltpu.async_copy(
            stem_hbm.at[taps[t].at[pl.ds(cc * GCH, GCH)]], bufs[b], gsem[b])

    def _start_write(i, b):
        t, cc = chunks[i]
        return pltpu.async_copy(
            bufs[b], g_hbm.at[t, pl.ds(base + cc * GCH, GCH)], wsem[b])

    gdesc = [None, None]
    wdesc = [None, None]
    gdesc[0] = _start_gather(0, 0)
    for i in range(n):
        b = i % 2
        if i + 1 < n:
            nb = (i + 1) % 2
            if wdesc[nb] is not None:
                wdesc[nb].wait()
            gdesc[nb] = _start_gather(i + 1, nb)
        gdesc[b].wait()
        wdesc[b] = _start_write(i, b)
    wdesc[(n - 1) % 2].wait()
    wdesc[n % 2].wait()


def _gather(stem_full, bi, yi, xi):
    mesh = plsc.VectorSubcoreMesh(core_axis_name="c", subcore_axis_name="s")
    return pl.kernel(
        _gather_body,
        out_type=jax.ShapeDtypeStruct((9, K, CH), jnp.int32),
        mesh=mesh,
        scratch_types=(
            [pltpu.VMEM((BPW,), jnp.int32) for _ in range(3)]
            + [pltpu.VMEM((BPW,), jnp.int32) for _ in range(9)]
            + [pltpu.VMEM((GCH, CH), jnp.int32) for _ in range(2)]
            + [pltpu.SemaphoreType.DMA for _ in range(4)]
        ),
    )(stem_full, bi, yi, xi)


def _head_body(g_ref, wc_ref, wr_ref, cb_ref, rb_ref, wro_ref, wcb_ref,
               hb_ref, o_ref):
    accc = jnp.zeros((KB, C), jnp.float32) + cb_ref[...]
    accr = jnp.zeros((KB, C), jnp.float32) + rb_ref[...]
    for t in range(9):
        gu = lax.bitcast_convert_type(g_ref[t], jnp.uint32)
        glo = lax.bitcast_convert_type(gu << 16, jnp.float32)
        ghi = lax.bitcast_convert_type(gu & jnp.uint32(0xFFFF0000), jnp.float32)
        wc = wc_ref[t]
        wr = wr_ref[t]
        accc += lax.dot_general(glo, wc[:, :CH], (((1,), (1,)), ((), ())),
                                preferred_element_type=jnp.float32)
        accc += lax.dot_general(ghi, wc[:, CH:], (((1,), (1,)), ((), ())),
                                preferred_element_type=jnp.float32)
        accr += lax.dot_general(glo, wr[:, :CH], (((1,), (1,)), ((), ())),
                                preferred_element_type=jnp.float32)
        accr += lax.dot_general(ghi, wr[:, CH:], (((1,), (1,)), ((), ())),
                                preferred_element_type=jnp.float32)
    cls_f = accc * jax.nn.sigmoid(accc)
    reg_f = accr * jax.nn.sigmoid(accr)
    out = lax.dot_general(reg_f, wro_ref[...], (((1,), (1,)), ((), ())),
                          preferred_element_type=jnp.float32)
    out += lax.dot_general(cls_f, wcb_ref[...], (((1,), (1,)), ((), ())),
                           preferred_element_type=jnp.float32)
    o_ref[...] = out + hb_ref[...]


def _heads(g, w9c, w9r, cb2, rb2, wro, wcb, hbias):
    return pl.pallas_call(
        _head_body,
        grid=(K // KB,),
        in_specs=[
            pl.BlockSpec((9, KB, CH), lambda i: (0, i, 0)),
            pl.BlockSpec((9, C, CP), lambda i: (0, 0, 0)),
            pl.BlockSpec((9, C, CP), lambda i: (0, 0, 0)),
            pl.BlockSpec((1, C), lambda i: (0, 0)),
            pl.BlockSpec((1, C), lambda i: (0, 0)),
            pl.BlockSpec((OUT_W, C), lambda i: (0, 0)),
            pl.BlockSpec((OUT_W, C), lambda i: (0, 0)),
            pl.BlockSpec((1, OUT_W), lambda i: (0, 0)),
        ],
        out_specs=pl.BlockSpec((KB, OUT_W), lambda i: (i, 0)),
        out_shape=jax.ShapeDtypeStruct((K, OUT_W), jnp.float32),
    )(g, w9c, w9r, cb2, rb2, wro, wcb, hbias)


def kernel(x, indices, stem_w, stem_b, cls_conv_w, cls_conv_b,
           reg_conv_w, reg_conv_b, cls_pred_w, cls_pred_b,
           reg_pred_w, reg_pred_b, obj_pred_w, obj_pred_b):
    xt = jnp.transpose(x.reshape(BS, C, NY * NX), (0, 2, 1)).reshape(NPIX, C)
    stem_wt = jnp.pad(stem_w, ((0, CP - C), (0, 0))).T
    stem_bp = jnp.pad(stem_b, (0, CP - C)).reshape(1, CP)
    stem_full = _stem(xt, stem_wt, stem_bp)

    idx32 = indices.astype(jnp.int32)
    g = _gather(stem_full, idx32[:, 0], idx32[:, 1], idx32[:, 2])

    w9c = jnp.pad(jnp.transpose(cls_conv_w.reshape(C, C, 9), (2, 0, 1)),
                  ((0, 0), (0, 0), (0, CP - C)))
    w9r = jnp.pad(jnp.transpose(reg_conv_w.reshape(C, C, 9), (2, 0, 1)),
                  ((0, 0), (0, 0), (0, CP - C)))
    wro = jnp.concatenate(
        [reg_pred_w, obj_pred_w, jnp.zeros((NC, C), jnp.float32)], axis=0)
    wcb = jnp.concatenate(
        [jnp.zeros((4 + 1, C), jnp.float32), cls_pred_w], axis=0)
    hbias = jnp.concatenate(
        [reg_pred_b, obj_pred_b, cls_pred_b]).reshape(1, OUT_W)

    return _heads(g, w9c, w9r, cls_conv_b.reshape(1, C),
                  reg_conv_b.reshape(1, C), wro, wcb, hbias)
